# baseline (speedup 1.0000x reference)
import functools

import jax
import jax.numpy as jnp
from jax import lax
from jax.experimental import pallas as pl
from jax.experimental.pallas import tpu as pltpu
from jax.experimental.pallas import tpu_sc as plsc

N = 8192
F = 8192
V = 6000
L = 16
NC, NS = 2, 16
NW = NC * NS
FPT = F // NW
TN = 256
EPS = 0.003
SHIFT = 0.5


def _face_body(oc_hbm, on_hbm, f_hbm, a_hbm, d_hbm, ct, nt, fv, av, dv):
    wid = lax.axis_index("c") * NS + lax.axis_index("s")
    base = wid * FPT
    pltpu.sync_copy(oc_hbm, ct)
    pltpu.sync_copy(on_hbm, nt)
    pltpu.sync_copy(f_hbm.at[pl.ds(base * 3, FPT * 3)], fv)

    zeros = jnp.zeros((L,), jnp.float32)
    ones = jnp.ones((L,), jnp.float32)
    lane3 = lax.iota(jnp.int32, L) * 3

    for k in range(FPT // L):
        sl = pl.ds(k * L, L)
        i0 = plsc.load_gather(fv, [lane3 + (k * 3 * L)]) * 3
        i1 = plsc.load_gather(fv, [lane3 + (k * 3 * L + 1)]) * 3
        i2 = plsc.load_gather(fv, [lane3 + (k * 3 * L + 2)]) * 3

        def g(tab, idx, comp):
            return plsc.load_gather(tab, [idx + comp])

        cx = (g(ct, i0, 0) + g(ct, i1, 0) + g(ct, i2, 0)) / 3.0 - SHIFT
        cy = (g(ct, i0, 1) + g(ct, i1, 1) + g(ct, i2, 1)) / 3.0 - SHIFT
        cz = (g(ct, i0, 2) + g(ct, i1, 2) + g(ct, i2, 2)) / 3.0 - SHIFT
        av[pl.ds(0 * FPT + k * L, L)] = -2.0 * cx
        av[pl.ds(1 * FPT + k * L, L)] = -2.0 * cy
        av[pl.ds(2 * FPT + k * L, L)] = -2.0 * cz
        av[pl.ds(3 * FPT + k * L, L)] = cx * cx + cy * cy + cz * cz
        av[pl.ds(4 * FPT + k * L, L)] = zeros
        av[pl.ds(5 * FPT + k * L, L)] = zeros
        av[pl.ds(6 * FPT + k * L, L)] = zeros
        av[pl.ds(7 * FPT + k * L, L)] = zeros

        p0x = g(nt, i0, 0)
        p0y = g(nt, i0, 1)
        p0z = g(nt, i0, 2)
        p1x = g(nt, i1, 0)
        p1y = g(nt, i1, 1)
        p1z = g(nt, i1, 2)
        p2x = g(nt, i2, 0)
        p2y = g(nt, i2, 1)
        p2z = g(nt, i2, 2)
        v1x = p1x - p0x
        v1y = p1y - p0y
        v1z = p1z - p0z
        v2x = p2x - p0x
        v2y = p2y - p0y
        v2z = p2z - p0z
        nx = v1y * v2z - v1z * v2y
        ny = v1z * v2x - v1x * v2z
        nz = v1x * v2y - v1y * v2x
        ctrx = (p0x + p1x + p2x) / 3.0
        ctry = (p0y + p1y + p2y) / 3.0
        ctrz = (p0z + p1z + p2z) / 3.0
        dv[pl.ds(0 * FPT + k * L, L)] = nx
        dv[pl.ds(1 * FPT + k * L, L)] = ny
        dv[pl.ds(2 * FPT + k * L, L)] = nz
        dv[pl.ds(3 * FPT + k * L, L)] = nx * ctrx + ny * ctry + nz * ctrz
        dv[pl.ds(4 * FPT + k * L, L)] = nx * nx + ny * ny + nz * nz
        dv[pl.ds(5 * FPT + k * L, L)] = ones
        dv[pl.ds(6 * FPT + k * L, L)] = zeros
        dv[pl.ds(7 * FPT + k * L, L)] = zeros

    for r in range(8):
        pltpu.sync_copy(av.at[pl.ds(r * FPT, FPT)], a_hbm.at[pl.ds(r * F + base, FPT)])
        pltpu.sync_copy(dv.at[pl.ds(r * FPT, FPT)], d_hbm.at[pl.ds(r * F + base, FPT)])


_face_tables_cache = []


def _face_tables(*args):
    if not _face_tables_cache:
        _face_tables_cache.append(pl.kernel(
            _face_body,
            out_type=(
                jax.ShapeDtypeStruct((8 * F,), jnp.float32),
                jax.ShapeDtypeStruct((8 * F,), jnp.float32),
            ),
            mesh=plsc.VectorSubcoreMesh(core_axis_name="c", subcore_axis_name="s"),
            scratch_types=[
                pltpu.VMEM((3 * V,), jnp.float32),
                pltpu.VMEM((3 * V,), jnp.float32),
                pltpu.VMEM((3 * FPT,), jnp.int32),
                pltpu.VMEM((8 * FPT,), jnp.float32),
                pltpu.VMEM((8 * FPT,), jnp.float32),
            ],
            compiler_params=pltpu.CompilerParams(needs_layout_passes=False),
        ))
    return _face_tables_cache[0](*args)


def _split3(x):
    f32, bf16 = jnp.float32, jnp.bfloat16
    x1 = x.astype(bf16)
    r1 = x - x1.astype(f32)
    x2 = r1.astype(bf16)
    r2 = r1 - x2.astype(f32)
    x3 = r2.astype(bf16)
    return x1, x2, x3


def _tc_body(c_ref, n_ref, a_ref, d_ref, o_ref, rhs_s, dns_s):
    i = pl.program_id(0)
    f32 = jnp.float32

    @pl.when(i == 0)
    def _():
        a1, a2, a3 = _split3(a_ref[...])
        rhs_s[...] = jnp.concatenate([a1, a2, a1, a3, a2, a1], axis=0)
        d = d_ref[...]
        w = 1.0 / (jnp.sqrt(d[4:5, :]) + 1e-8)
        dn = jnp.concatenate(
            [d[0:3, :] * w, d[3:4, :] * w, jnp.zeros((4, F), f32)], axis=0)
        dn1 = dn.astype(jnp.bfloat16)
        dn2 = (dn - dn1.astype(f32)).astype(jnp.bfloat16)
        dns_s[...] = jnp.concatenate([dn1, dn2, dn1], axis=0)

    c = c_ref[...]
    cs = jnp.concatenate(
        [c - SHIFT, jnp.ones((TN, 1), f32), jnp.zeros((TN, 4), f32)], axis=1)
    c1, c2, c3 = _split3(cs)
    lhs = jnp.concatenate([c1, c1, c2, c1, c2, c3], axis=1)
    scores = lax.dot_general(
        lhs, rhs_s[...], (((1,), (0,)), ((), ())),
        preferred_element_type=f32,
    )

    pa = jnp.concatenate(
        [n_ref[...], -jnp.ones((TN, 1), f32), jnp.zeros((TN, 4), f32)], axis=1)
    p1 = pa.astype(jnp.bfloat16)
    p2 = (pa - p1.astype(f32)).astype(jnp.bfloat16)
    pl_lhs = jnp.concatenate([p1, p1, p2], axis=1)
    svals = lax.dot_general(
        pl_lhs, dns_s[...], (((1,), (0,)), ((), ())),
        preferred_element_type=f32,
    )

    rowmin = jnp.min(scores, axis=1, keepdims=True)
    eq = scores == rowmin
    dist = jnp.min(jnp.where(eq, svals, jnp.float32(1e30)), axis=1)
    pen = jnp.maximum(EPS - dist, 0.0)
    contrib = jnp.sum(pen * pen * pen)

    @pl.when(i == 0)
    def _():
        o_ref[...] = jnp.zeros((1, 1), jnp.float32)

    o_ref[...] += contrib.reshape(1, 1)


_penalty_call = pl.pallas_call(
    _tc_body,
    grid=(N // TN,),
    in_specs=[
        pl.BlockSpec((TN, 3), lambda i: (i, 0)),
        pl.BlockSpec((TN, 3), lambda i: (i, 0)),
        pl.BlockSpec((8, F), lambda i: (0, 0)),
        pl.BlockSpec((8, F), lambda i: (0, 0)),
    ],
    out_specs=pl.BlockSpec((1, 1), lambda i: (0, 0)),
    out_shape=jax.ShapeDtypeStruct((1, 1), jnp.float32),
    scratch_shapes=[
        pltpu.VMEM((48, F), jnp.bfloat16),
        pltpu.VMEM((24, F), jnp.bfloat16),
    ],
)


def kernel(next_pos, curr_pos, obstacle_next_pos, obstacle_curr_pos, obstacle_faces):
    faces = obstacle_faces.astype(jnp.int32)
    a_t, d_t = _face_tables(
        obstacle_curr_pos.reshape(-1),
        obstacle_next_pos.reshape(-1),
        faces.reshape(-1),
    )
    out = _penalty_call(curr_pos, next_pos, a_t.reshape(8, F), d_t.reshape(8, F))
    return out[0, 0]


# speedup vs baseline: 1.0022x; 1.0022x over previous
import functools

import jax
import jax.numpy as jnp
from jax import lax
from jax.experimental import pallas as pl
from jax.experimental.pallas import tpu as pltpu
from jax.experimental.pallas import tpu_sc as plsc

N = 8192
F = 8192
V = 6000
L = 16
NC, NS = 2, 16
NW = NC * NS
FPT = F // NW
TN = 256
EPS = 0.003
SHIFT = 0.5


def _face_body(oc_hbm, on_hbm, f_hbm, a_hbm, d_hbm, ct, nt, fv, av, dv):
    wid = lax.axis_index("c") * NS + lax.axis_index("s")
    base = wid * FPT
    pltpu.sync_copy(oc_hbm, ct)
    pltpu.sync_copy(on_hbm, nt)
    pltpu.sync_copy(f_hbm.at[pl.ds(base * 3, FPT * 3)], fv)

    zeros = jnp.zeros((L,), jnp.float32)
    ones = jnp.ones((L,), jnp.float32)
    lane3 = lax.iota(jnp.int32, L) * 3

    for k in range(FPT // L):
        sl = pl.ds(k * L, L)
        i0 = plsc.load_gather(fv, [lane3 + (k * 3 * L)]) * 3
        i1 = plsc.load_gather(fv, [lane3 + (k * 3 * L + 1)]) * 3
        i2 = plsc.load_gather(fv, [lane3 + (k * 3 * L + 2)]) * 3

        def g(tab, idx, comp):
            return plsc.load_gather(tab, [idx + comp])

        cx = (g(ct, i0, 0) + g(ct, i1, 0) + g(ct, i2, 0)) / 3.0 - SHIFT
        cy = (g(ct, i0, 1) + g(ct, i1, 1) + g(ct, i2, 1)) / 3.0 - SHIFT
        cz = (g(ct, i0, 2) + g(ct, i1, 2) + g(ct, i2, 2)) / 3.0 - SHIFT
        av[pl.ds(0 * FPT + k * L, L)] = -2.0 * cx
        av[pl.ds(1 * FPT + k * L, L)] = -2.0 * cy
        av[pl.ds(2 * FPT + k * L, L)] = -2.0 * cz
        av[pl.ds(3 * FPT + k * L, L)] = cx * cx + cy * cy + cz * cz
        av[pl.ds(4 * FPT + k * L, L)] = zeros
        av[pl.ds(5 * FPT + k * L, L)] = zeros
        av[pl.ds(6 * FPT + k * L, L)] = zeros
        av[pl.ds(7 * FPT + k * L, L)] = zeros

        p0x = g(nt, i0, 0)
        p0y = g(nt, i0, 1)
        p0z = g(nt, i0, 2)
        p1x = g(nt, i1, 0)
        p1y = g(nt, i1, 1)
        p1z = g(nt, i1, 2)
        p2x = g(nt, i2, 0)
        p2y = g(nt, i2, 1)
        p2z = g(nt, i2, 2)
        v1x = p1x - p0x
        v1y = p1y - p0y
        v1z = p1z - p0z
        v2x = p2x - p0x
        v2y = p2y - p0y
        v2z = p2z - p0z
        nx = v1y * v2z - v1z * v2y
        ny = v1z * v2x - v1x * v2z
        nz = v1x * v2y - v1y * v2x
        ctrx = (p0x + p1x + p2x) / 3.0
        ctry = (p0y + p1y + p2y) / 3.0
        ctrz = (p0z + p1z + p2z) / 3.0
        dv[pl.ds(0 * FPT + k * L, L)] = nx
        dv[pl.ds(1 * FPT + k * L, L)] = ny
        dv[pl.ds(2 * FPT + k * L, L)] = nz
        dv[pl.ds(3 * FPT + k * L, L)] = nx * ctrx + ny * ctry + nz * ctrz
        dv[pl.ds(4 * FPT + k * L, L)] = nx * nx + ny * ny + nz * nz
        dv[pl.ds(5 * FPT + k * L, L)] = ones
        dv[pl.ds(6 * FPT + k * L, L)] = zeros
        dv[pl.ds(7 * FPT + k * L, L)] = zeros

    pltpu.sync_copy(av, a_hbm.at[wid])
    pltpu.sync_copy(dv, d_hbm.at[wid])


_face_tables_cache = []


def _face_tables(*args):
    if not _face_tables_cache:
        _face_tables_cache.append(pl.kernel(
            _face_body,
            out_type=(
                jax.ShapeDtypeStruct((NW, 8 * FPT), jnp.float32),
                jax.ShapeDtypeStruct((NW, 8 * FPT), jnp.float32),
            ),
            mesh=plsc.VectorSubcoreMesh(core_axis_name="c", subcore_axis_name="s"),
            scratch_types=[
                pltpu.VMEM((3 * V,), jnp.float32),
                pltpu.VMEM((3 * V,), jnp.float32),
                pltpu.VMEM((3 * FPT,), jnp.int32),
                pltpu.VMEM((8 * FPT,), jnp.float32),
                pltpu.VMEM((8 * FPT,), jnp.float32),
            ],
            compiler_params=pltpu.CompilerParams(needs_layout_passes=False),
        ))
    return _face_tables_cache[0](*args)


def _split3(x):
    f32, bf16 = jnp.float32, jnp.bfloat16
    x1 = x.astype(bf16)
    r1 = x - x1.astype(f32)
    x2 = r1.astype(bf16)
    r2 = r1 - x2.astype(f32)
    x3 = r2.astype(bf16)
    return x1, x2, x3


def _tc_body(c_ref, n_ref, a_ref, d_ref, o_ref, rhs_s, dns_s):
    i = pl.program_id(0)
    f32 = jnp.float32

    @pl.when(i == 0)
    def _():
        a1, a2, a3 = _split3(a_ref[...])
        rhs_s[...] = jnp.concatenate([a1, a2, a1, a3, a2, a1], axis=0)
        d = d_ref[...]
        w = 1.0 / (jnp.sqrt(d[4:5, :]) + 1e-8)
        dn = jnp.concatenate(
            [d[0:3, :] * w, d[3:4, :] * w, jnp.zeros((4, F), f32)], axis=0)
        dn1 = dn.astype(jnp.bfloat16)
        dn2 = (dn - dn1.astype(f32)).astype(jnp.bfloat16)
        dns_s[...] = jnp.concatenate([dn1, dn2, dn1], axis=0)

    c = c_ref[...]
    cs = jnp.concatenate(
        [c - SHIFT, jnp.ones((TN, 1), f32), jnp.zeros((TN, 4), f32)], axis=1)
    c1, c2, c3 = _split3(cs)
    lhs = jnp.concatenate([c1, c1, c2, c1, c2, c3], axis=1)
    scores = lax.dot_general(
        lhs, rhs_s[...], (((1,), (0,)), ((), ())),
        preferred_element_type=f32,
    )

    pa = jnp.concatenate(
        [n_ref[...], -jnp.ones((TN, 1), f32), jnp.zeros((TN, 4), f32)], axis=1)
    p1 = pa.astype(jnp.bfloat16)
    p2 = (pa - p1.astype(f32)).astype(jnp.bfloat16)
    pl_lhs = jnp.concatenate([p1, p1, p2], axis=1)
    svals = lax.dot_general(
        pl_lhs, dns_s[...], (((1,), (0,)), ((), ())),
        preferred_element_type=f32,
    )

    rowmin = jnp.min(scores, axis=1, keepdims=True)
    eq = scores == rowmin
    dist = jnp.min(jnp.where(eq, svals, jnp.float32(1e30)), axis=1)
    pen = jnp.maximum(EPS - dist, 0.0)
    contrib = jnp.sum(pen * pen * pen)

    @pl.when(i == 0)
    def _():
        o_ref[...] = jnp.zeros((1, 1), jnp.float32)

    o_ref[...] += contrib.reshape(1, 1)


_penalty_call = pl.pallas_call(
    _tc_body,
    grid=(N // TN,),
    in_specs=[
        pl.BlockSpec((TN, 3), lambda i: (i, 0)),
        pl.BlockSpec((TN, 3), lambda i: (i, 0)),
        pl.BlockSpec((8, F), lambda i: (0, 0)),
        pl.BlockSpec((8, F), lambda i: (0, 0)),
    ],
    out_specs=pl.BlockSpec((1, 1), lambda i: (0, 0)),
    out_shape=jax.ShapeDtypeStruct((1, 1), jnp.float32),
    scratch_shapes=[
        pltpu.VMEM((48, F), jnp.bfloat16),
        pltpu.VMEM((24, F), jnp.bfloat16),
    ],
)


def kernel(next_pos, curr_pos, obstacle_next_pos, obstacle_curr_pos, obstacle_faces):
    faces = obstacle_faces.astype(jnp.int32)
    a_t, d_t = _face_tables(
        obstacle_curr_pos.reshape(-1),
        obstacle_next_pos.reshape(-1),
        faces.reshape(-1),
    )
    a_mat = a_t.reshape(NW, 8, FPT).transpose(1, 0, 2).reshape(8, F)
    d_mat = d_t.reshape(NW, 8, FPT).transpose(1, 0, 2).reshape(8, F)
    out = _penalty_call(curr_pos, next_pos, a_mat, d_mat)
    return out[0, 0]


# speedup vs baseline: 1.0763x; 1.0740x over previous
import functools

import jax
import jax.numpy as jnp
from jax import lax
from jax.experimental import pallas as pl
from jax.experimental.pallas import tpu as pltpu
from jax.experimental.pallas import tpu_sc as plsc

N = 8192
F = 8192
V = 6000
L = 16
NC, NS = 2, 16
NW = NC * NS
FPT = F // NW
TN = 256
EPS = 0.003
SHIFT = 0.5


def _face_body(oc_hbm, on_hbm, f_hbm, a_hbm, d_hbm, ct, nt, fv0, fv1, fv2, av, dv):
    wid = lax.axis_index("c") * NS + lax.axis_index("s")
    base = wid * FPT
    pltpu.sync_copy(oc_hbm, ct)
    pltpu.sync_copy(on_hbm, nt)
    for c, fv in ((0, fv0), (1, fv1), (2, fv2)):
        pltpu.sync_copy(f_hbm.at[pl.ds(c * F + base, FPT)], fv)

    zeros = jnp.zeros((L,), jnp.float32)
    ones = jnp.ones((L,), jnp.float32)

    for k in range(FPT // L):
        sl = pl.ds(k * L, L)
        i0 = fv0[sl]
        i1 = fv1[sl]
        i2 = fv2[sl]

        def g(tab, idx, comp):
            return plsc.load_gather(tab, [idx + comp * V])

        cx = (g(ct, i0, 0) + g(ct, i1, 0) + g(ct, i2, 0)) / 3.0 - SHIFT
        cy = (g(ct, i0, 1) + g(ct, i1, 1) + g(ct, i2, 1)) / 3.0 - SHIFT
        cz = (g(ct, i0, 2) + g(ct, i1, 2) + g(ct, i2, 2)) / 3.0 - SHIFT
        av[pl.ds(0 * FPT + k * L, L)] = -2.0 * cx
        av[pl.ds(1 * FPT + k * L, L)] = -2.0 * cy
        av[pl.ds(2 * FPT + k * L, L)] = -2.0 * cz
        av[pl.ds(3 * FPT + k * L, L)] = cx * cx + cy * cy + cz * cz
        av[pl.ds(4 * FPT + k * L, L)] = zeros
        av[pl.ds(5 * FPT + k * L, L)] = zeros
        av[pl.ds(6 * FPT + k * L, L)] = zeros
        av[pl.ds(7 * FPT + k * L, L)] = zeros

        p0x = g(nt, i0, 0)
        p0y = g(nt, i0, 1)
        p0z = g(nt, i0, 2)
        p1x = g(nt, i1, 0)
        p1y = g(nt, i1, 1)
        p1z = g(nt, i1, 2)
        p2x = g(nt, i2, 0)
        p2y = g(nt, i2, 1)
        p2z = g(nt, i2, 2)
        v1x = p1x - p0x
        v1y = p1y - p0y
        v1z = p1z - p0z
        v2x = p2x - p0x
        v2y = p2y - p0y
        v2z = p2z - p0z
        nx = v1y * v2z - v1z * v2y
        ny = v1z * v2x - v1x * v2z
        nz = v1x * v2y - v1y * v2x
        ctrx = (p0x + p1x + p2x) / 3.0
        ctry = (p0y + p1y + p2y) / 3.0
        ctrz = (p0z + p1z + p2z) / 3.0
        dv[pl.ds(0 * FPT + k * L, L)] = nx
        dv[pl.ds(1 * FPT + k * L, L)] = ny
        dv[pl.ds(2 * FPT + k * L, L)] = nz
        dv[pl.ds(3 * FPT + k * L, L)] = nx * ctrx + ny * ctry + nz * ctrz
        dv[pl.ds(4 * FPT + k * L, L)] = nx * nx + ny * ny + nz * nz
        dv[pl.ds(5 * FPT + k * L, L)] = ones
        dv[pl.ds(6 * FPT + k * L, L)] = zeros
        dv[pl.ds(7 * FPT + k * L, L)] = zeros

    pltpu.sync_copy(av, a_hbm.at[wid])
    pltpu.sync_copy(dv, d_hbm.at[wid])


_face_tables_cache = []


def _face_tables(*args):
    if not _face_tables_cache:
        _face_tables_cache.append(pl.kernel(
            _face_body,
            out_type=(
                jax.ShapeDtypeStruct((NW, 8 * FPT), jnp.float32),
                jax.ShapeDtypeStruct((NW, 8 * FPT), jnp.float32),
            ),
            mesh=plsc.VectorSubcoreMesh(core_axis_name="c", subcore_axis_name="s"),
            scratch_types=[
                pltpu.VMEM((3 * V,), jnp.float32),
                pltpu.VMEM((3 * V,), jnp.float32),
                pltpu.VMEM((FPT,), jnp.int32),
                pltpu.VMEM((FPT,), jnp.int32),
                pltpu.VMEM((FPT,), jnp.int32),
                pltpu.VMEM((8 * FPT,), jnp.float32),
                pltpu.VMEM((8 * FPT,), jnp.float32),
            ],
            compiler_params=pltpu.CompilerParams(needs_layout_passes=False),
        ))
    return _face_tables_cache[0](*args)


def _split3(x):
    f32, bf16 = jnp.float32, jnp.bfloat16
    x1 = x.astype(bf16)
    r1 = x - x1.astype(f32)
    x2 = r1.astype(bf16)
    r2 = r1 - x2.astype(f32)
    x3 = r2.astype(bf16)
    return x1, x2, x3


def _tc_body(c_ref, n_ref, a_ref, d_ref, o_ref, rhs_s, dns_s):
    i = pl.program_id(0)
    f32 = jnp.float32

    @pl.when(i == 0)
    def _():
        a1, a2, a3 = _split3(a_ref[...])
        rhs_s[...] = jnp.concatenate([a1, a2, a1, a3, a2, a1], axis=0)
        d = d_ref[...]
        w = 1.0 / (jnp.sqrt(d[4:5, :]) + 1e-8)
        dn = jnp.concatenate(
            [d[0:3, :] * w, d[3:4, :] * w, jnp.zeros((4, F), f32)], axis=0)
        dn1 = dn.astype(jnp.bfloat16)
        dn2 = (dn - dn1.astype(f32)).astype(jnp.bfloat16)
        dns_s[...] = jnp.concatenate([dn1, dn2, dn1], axis=0)

    c = c_ref[...]
    cs = jnp.concatenate(
        [c - SHIFT, jnp.ones((TN, 1), f32), jnp.zeros((TN, 4), f32)], axis=1)
    c1, c2, c3 = _split3(cs)
    lhs = jnp.concatenate([c1, c1, c2, c1, c2, c3], axis=1)
    scores = lax.dot_general(
        lhs, rhs_s[...], (((1,), (0,)), ((), ())),
        preferred_element_type=f32,
    )

    pa = jnp.concatenate(
        [n_ref[...], -jnp.ones((TN, 1), f32), jnp.zeros((TN, 4), f32)], axis=1)
    p1 = pa.astype(jnp.bfloat16)
    p2 = (pa - p1.astype(f32)).astype(jnp.bfloat16)
    pl_lhs = jnp.concatenate([p1, p1, p2], axis=1)
    svals = lax.dot_general(
        pl_lhs, dns_s[...], (((1,), (0,)), ((), ())),
        preferred_element_type=f32,
    )

    rowmin = jnp.min(scores, axis=1, keepdims=True)
    eq = scores == rowmin
    dist = jnp.min(jnp.where(eq, svals, jnp.float32(1e30)), axis=1)
    pen = jnp.maximum(EPS - dist, 0.0)
    contrib = jnp.sum(pen * pen * pen)

    @pl.when(i == 0)
    def _():
        o_ref[...] = jnp.zeros((1, 1), jnp.float32)

    o_ref[...] += contrib.reshape(1, 1)


_penalty_call = pl.pallas_call(
    _tc_body,
    grid=(N // TN,),
    in_specs=[
        pl.BlockSpec((TN, 3), lambda i: (i, 0)),
        pl.BlockSpec((TN, 3), lambda i: (i, 0)),
        pl.BlockSpec((8, F), lambda i: (0, 0)),
        pl.BlockSpec((8, F), lambda i: (0, 0)),
    ],
    out_specs=pl.BlockSpec((1, 1), lambda i: (0, 0)),
    out_shape=jax.ShapeDtypeStruct((1, 1), jnp.float32),
    scratch_shapes=[
        pltpu.VMEM((48, F), jnp.bfloat16),
        pltpu.VMEM((24, F), jnp.bfloat16),
    ],
)


def kernel(next_pos, curr_pos, obstacle_next_pos, obstacle_curr_pos, obstacle_faces):
    faces = obstacle_faces.astype(jnp.int32)
    a_t, d_t = _face_tables(
        obstacle_curr_pos.T.reshape(-1),
        obstacle_next_pos.T.reshape(-1),
        faces.T.reshape(-1),
    )
    a_mat = a_t.reshape(NW, 8, FPT).transpose(1, 0, 2).reshape(8, F)
    d_mat = d_t.reshape(NW, 8, FPT).transpose(1, 0, 2).reshape(8, F)
    out = _penalty_call(curr_pos, next_pos, a_mat, d_mat)
    return out[0, 0]


# speedup vs baseline: 1.1135x; 1.0345x over previous
import functools

import jax
import jax.numpy as jnp
from jax import lax
from jax.experimental import pallas as pl
from jax.experimental.pallas import tpu as pltpu
from jax.experimental.pallas import tpu_sc as plsc

N = 8192
F = 8192
V = 6000
L = 16
NC, NS = 2, 16
NW = NC * NS
FPT = F // NW
TN = 512
EPS = 0.003
SHIFT = 0.5


def _face_body(oc_hbm, on_hbm, f_hbm, a_hbm, d_hbm, ct, nt, fv0, fv1, fv2, av, dv):
    wid = lax.axis_index("c") * NS + lax.axis_index("s")
    base = wid * FPT
    pltpu.sync_copy(oc_hbm, ct)
    pltpu.sync_copy(on_hbm, nt)
    for c, fv in ((0, fv0), (1, fv1), (2, fv2)):
        pltpu.sync_copy(f_hbm.at[pl.ds(c * F + base, FPT)], fv)

    zeros = jnp.zeros((L,), jnp.float32)
    ones = jnp.ones((L,), jnp.float32)

    for k in range(FPT // L):
        sl = pl.ds(k * L, L)
        i0 = fv0[sl]
        i1 = fv1[sl]
        i2 = fv2[sl]

        def g(tab, idx, comp):
            return plsc.load_gather(tab, [idx + comp * V])

        cx = (g(ct, i0, 0) + g(ct, i1, 0) + g(ct, i2, 0)) / 3.0 - SHIFT
        cy = (g(ct, i0, 1) + g(ct, i1, 1) + g(ct, i2, 1)) / 3.0 - SHIFT
        cz = (g(ct, i0, 2) + g(ct, i1, 2) + g(ct, i2, 2)) / 3.0 - SHIFT
        av[pl.ds(0 * FPT + k * L, L)] = -2.0 * cx
        av[pl.ds(1 * FPT + k * L, L)] = -2.0 * cy
        av[pl.ds(2 * FPT + k * L, L)] = -2.0 * cz
        av[pl.ds(3 * FPT + k * L, L)] = cx * cx + cy * cy + cz * cz
        av[pl.ds(4 * FPT + k * L, L)] = zeros
        av[pl.ds(5 * FPT + k * L, L)] = zeros
        av[pl.ds(6 * FPT + k * L, L)] = zeros
        av[pl.ds(7 * FPT + k * L, L)] = zeros

        p0x = g(nt, i0, 0)
        p0y = g(nt, i0, 1)
        p0z = g(nt, i0, 2)
        p1x = g(nt, i1, 0)
        p1y = g(nt, i1, 1)
        p1z = g(nt, i1, 2)
        p2x = g(nt, i2, 0)
        p2y = g(nt, i2, 1)
        p2z = g(nt, i2, 2)
        v1x = p1x - p0x
        v1y = p1y - p0y
        v1z = p1z - p0z
        v2x = p2x - p0x
        v2y = p2y - p0y
        v2z = p2z - p0z
        nx = v1y * v2z - v1z * v2y
        ny = v1z * v2x - v1x * v2z
        nz = v1x * v2y - v1y * v2x
        ctrx = (p0x + p1x + p2x) / 3.0
        ctry = (p0y + p1y + p2y) / 3.0
        ctrz = (p0z + p1z + p2z) / 3.0
        dv[pl.ds(0 * FPT + k * L, L)] = nx
        dv[pl.ds(1 * FPT + k * L, L)] = ny
        dv[pl.ds(2 * FPT + k * L, L)] = nz
        dv[pl.ds(3 * FPT + k * L, L)] = nx * ctrx + ny * ctry + nz * ctrz
        dv[pl.ds(4 * FPT + k * L, L)] = nx * nx + ny * ny + nz * nz
        dv[pl.ds(5 * FPT + k * L, L)] = ones
        dv[pl.ds(6 * FPT + k * L, L)] = zeros
        dv[pl.ds(7 * FPT + k * L, L)] = zeros

    pltpu.sync_copy(av, a_hbm.at[wid])
    pltpu.sync_copy(dv, d_hbm.at[wid])


_face_tables_cache = []


def _face_tables(*args):
    if not _face_tables_cache:
        _face_tables_cache.append(pl.kernel(
            _face_body,
            out_type=(
                jax.ShapeDtypeStruct((NW, 8 * FPT), jnp.float32),
                jax.ShapeDtypeStruct((NW, 8 * FPT), jnp.float32),
            ),
            mesh=plsc.VectorSubcoreMesh(core_axis_name="c", subcore_axis_name="s"),
            scratch_types=[
                pltpu.VMEM((3 * V,), jnp.float32),
                pltpu.VMEM((3 * V,), jnp.float32),
                pltpu.VMEM((FPT,), jnp.int32),
                pltpu.VMEM((FPT,), jnp.int32),
                pltpu.VMEM((FPT,), jnp.int32),
                pltpu.VMEM((8 * FPT,), jnp.float32),
                pltpu.VMEM((8 * FPT,), jnp.float32),
            ],
            compiler_params=pltpu.CompilerParams(needs_layout_passes=False),
        ))
    return _face_tables_cache[0](*args)


def _split3(x):
    f32, bf16 = jnp.float32, jnp.bfloat16
    x1 = x.astype(bf16)
    r1 = x - x1.astype(f32)
    x2 = r1.astype(bf16)
    r2 = r1 - x2.astype(f32)
    x3 = r2.astype(bf16)
    return x1, x2, x3


def _tc_body(c_ref, n_ref, a_ref, d_ref, o_ref, rhs_s, dns_s):
    i = pl.program_id(0)
    f32 = jnp.float32

    @pl.when(i == 0)
    def _():
        a1, a2, a3 = _split3(a_ref[...])
        rhs_s[...] = jnp.concatenate([a1, a2, a1, a3, a2, a1], axis=0)
        d = d_ref[...]
        w = 1.0 / (jnp.sqrt(d[4:5, :]) + 1e-8)
        dn = jnp.concatenate(
            [d[0:3, :] * w, d[3:4, :] * w, jnp.zeros((4, F), f32)], axis=0)
        dn1 = dn.astype(jnp.bfloat16)
        dn2 = (dn - dn1.astype(f32)).astype(jnp.bfloat16)
        dns_s[...] = jnp.concatenate([dn1, dn2, dn1], axis=0)

    c = c_ref[...]
    cs = jnp.concatenate(
        [c - SHIFT, jnp.ones((TN, 1), f32), jnp.zeros((TN, 4), f32)], axis=1)
    c1, c2, c3 = _split3(cs)
    lhs = jnp.concatenate([c1, c1, c2, c1, c2, c3], axis=1)
    scores = lax.dot_general(
        lhs, rhs_s[...], (((1,), (0,)), ((), ())),
        preferred_element_type=f32,
    )

    pa = jnp.concatenate(
        [n_ref[...], -jnp.ones((TN, 1), f32), jnp.zeros((TN, 4), f32)], axis=1)
    p1 = pa.astype(jnp.bfloat16)
    p2 = (pa - p1.astype(f32)).astype(jnp.bfloat16)
    pl_lhs = jnp.concatenate([p1, p1, p2], axis=1)
    svals = lax.dot_general(
        pl_lhs, dns_s[...], (((1,), (0,)), ((), ())),
        preferred_element_type=f32,
    )

    rowmin = jnp.min(scores, axis=1, keepdims=True)
    eq = scores == rowmin
    dist = jnp.min(jnp.where(eq, svals, jnp.float32(1e30)), axis=1)
    pen = jnp.maximum(EPS - dist, 0.0)
    contrib = jnp.sum(pen * pen * pen)

    @pl.when(i == 0)
    def _():
        o_ref[...] = jnp.zeros((1, 1), jnp.float32)

    o_ref[...] += contrib.reshape(1, 1)


_penalty_call = pl.pallas_call(
    _tc_body,
    grid=(N // TN,),
    in_specs=[
        pl.BlockSpec((TN, 3), lambda i: (i, 0)),
        pl.BlockSpec((TN, 3), lambda i: (i, 0)),
        pl.BlockSpec((8, F), lambda i: (0, 0)),
        pl.BlockSpec((8, F), lambda i: (0, 0)),
    ],
    out_specs=pl.BlockSpec((1, 1), lambda i: (0, 0)),
    out_shape=jax.ShapeDtypeStruct((1, 1), jnp.float32),
    scratch_shapes=[
        pltpu.VMEM((48, F), jnp.bfloat16),
        pltpu.VMEM((24, F), jnp.bfloat16),
    ],
)


def kernel(next_pos, curr_pos, obstacle_next_pos, obstacle_curr_pos, obstacle_faces):
    faces = obstacle_faces.astype(jnp.int32)
    a_t, d_t = _face_tables(
        obstacle_curr_pos.T.reshape(-1),
        obstacle_next_pos.T.reshape(-1),
        faces.T.reshape(-1),
    )
    a_mat = a_t.reshape(NW, 8, FPT).transpose(1, 0, 2).reshape(8, F)
    d_mat = d_t.reshape(NW, 8, FPT).transpose(1, 0, 2).reshape(8, F)
    out = _penalty_call(curr_pos, next_pos, a_mat, d_mat)
    return out[0, 0]


# speedup vs baseline: 1.1517x; 1.0343x over previous
import functools

import jax
import jax.numpy as jnp
from jax import lax
from jax.experimental import pallas as pl
from jax.experimental.pallas import tpu as pltpu
from jax.experimental.pallas import tpu_sc as plsc

N = 8192
F = 8192
V = 6000
L = 16
NC, NS = 2, 16
NW = NC * NS
FPT = F // NW
TN = 1024
EPS = 0.003
SHIFT = 0.5


def _face_body(oc_hbm, on_hbm, f_hbm, a_hbm, d_hbm, ct, nt, fv0, fv1, fv2, av, dv):
    wid = lax.axis_index("c") * NS + lax.axis_index("s")
    base = wid * FPT
    pltpu.sync_copy(oc_hbm, ct)
    pltpu.sync_copy(on_hbm, nt)
    for c, fv in ((0, fv0), (1, fv1), (2, fv2)):
        pltpu.sync_copy(f_hbm.at[pl.ds(c * F + base, FPT)], fv)

    zeros = jnp.zeros((L,), jnp.float32)
    ones = jnp.ones((L,), jnp.float32)

    for k in range(FPT // L):
        sl = pl.ds(k * L, L)
        i0 = fv0[sl]
        i1 = fv1[sl]
        i2 = fv2[sl]

        def g(tab, idx, comp):
            return plsc.load_gather(tab, [idx + comp * V])

        cx = (g(ct, i0, 0) + g(ct, i1, 0) + g(ct, i2, 0)) / 3.0 - SHIFT
        cy = (g(ct, i0, 1) + g(ct, i1, 1) + g(ct, i2, 1)) / 3.0 - SHIFT
        cz = (g(ct, i0, 2) + g(ct, i1, 2) + g(ct, i2, 2)) / 3.0 - SHIFT
        av[pl.ds(0 * FPT + k * L, L)] = -2.0 * cx
        av[pl.ds(1 * FPT + k * L, L)] = -2.0 * cy
        av[pl.ds(2 * FPT + k * L, L)] = -2.0 * cz
        av[pl.ds(3 * FPT + k * L, L)] = cx * cx + cy * cy + cz * cz
        av[pl.ds(4 * FPT + k * L, L)] = zeros
        av[pl.ds(5 * FPT + k * L, L)] = zeros
        av[pl.ds(6 * FPT + k * L, L)] = zeros
        av[pl.ds(7 * FPT + k * L, L)] = zeros

        p0x = g(nt, i0, 0)
        p0y = g(nt, i0, 1)
        p0z = g(nt, i0, 2)
        p1x = g(nt, i1, 0)
        p1y = g(nt, i1, 1)
        p1z = g(nt, i1, 2)
        p2x = g(nt, i2, 0)
        p2y = g(nt, i2, 1)
        p2z = g(nt, i2, 2)
        v1x = p1x - p0x
        v1y = p1y - p0y
        v1z = p1z - p0z
        v2x = p2x - p0x
        v2y = p2y - p0y
        v2z = p2z - p0z
        nx = v1y * v2z - v1z * v2y
        ny = v1z * v2x - v1x * v2z
        nz = v1x * v2y - v1y * v2x
        ctrx = (p0x + p1x + p2x) / 3.0
        ctry = (p0y + p1y + p2y) / 3.0
        ctrz = (p0z + p1z + p2z) / 3.0
        dv[pl.ds(0 * FPT + k * L, L)] = nx
        dv[pl.ds(1 * FPT + k * L, L)] = ny
        dv[pl.ds(2 * FPT + k * L, L)] = nz
        dv[pl.ds(3 * FPT + k * L, L)] = nx * ctrx + ny * ctry + nz * ctrz
        dv[pl.ds(4 * FPT + k * L, L)] = nx * nx + ny * ny + nz * nz
        dv[pl.ds(5 * FPT + k * L, L)] = ones
        dv[pl.ds(6 * FPT + k * L, L)] = zeros
        dv[pl.ds(7 * FPT + k * L, L)] = zeros

    pltpu.sync_copy(av, a_hbm.at[wid])
    pltpu.sync_copy(dv, d_hbm.at[wid])


_face_tables_cache = []


def _face_tables(*args):
    if not _face_tables_cache:
        _face_tables_cache.append(pl.kernel(
            _face_body,
            out_type=(
                jax.ShapeDtypeStruct((NW, 8 * FPT), jnp.float32),
                jax.ShapeDtypeStruct((NW, 8 * FPT), jnp.float32),
            ),
            mesh=plsc.VectorSubcoreMesh(core_axis_name="c", subcore_axis_name="s"),
            scratch_types=[
                pltpu.VMEM((3 * V,), jnp.float32),
                pltpu.VMEM((3 * V,), jnp.float32),
                pltpu.VMEM((FPT,), jnp.int32),
                pltpu.VMEM((FPT,), jnp.int32),
                pltpu.VMEM((FPT,), jnp.int32),
                pltpu.VMEM((8 * FPT,), jnp.float32),
                pltpu.VMEM((8 * FPT,), jnp.float32),
            ],
            compiler_params=pltpu.CompilerParams(needs_layout_passes=False),
        ))
    return _face_tables_cache[0](*args)


def _split3(x):
    f32, bf16 = jnp.float32, jnp.bfloat16
    x1 = x.astype(bf16)
    r1 = x - x1.astype(f32)
    x2 = r1.astype(bf16)
    r2 = r1 - x2.astype(f32)
    x3 = r2.astype(bf16)
    return x1, x2, x3


def _tc_body(c_ref, n_ref, a_ref, d_ref, o_ref, rhs_s, dns_s):
    i = pl.program_id(0)
    f32 = jnp.float32

    @pl.when(i == 0)
    def _():
        a1, a2, a3 = _split3(a_ref[...])
        rhs_s[...] = jnp.concatenate([a1, a2, a1, a3, a2, a1], axis=0)
        d = d_ref[...]
        w = 1.0 / (jnp.sqrt(d[4:5, :]) + 1e-8)
        dn = jnp.concatenate(
            [d[0:3, :] * w, d[3:4, :] * w, jnp.zeros((4, F), f32)], axis=0)
        dn1 = dn.astype(jnp.bfloat16)
        dn2 = (dn - dn1.astype(f32)).astype(jnp.bfloat16)
        dns_s[...] = jnp.concatenate([dn1, dn2, dn1], axis=0)

    c = c_ref[...]
    cs = jnp.concatenate(
        [c - SHIFT, jnp.ones((TN, 1), f32), jnp.zeros((TN, 4), f32)], axis=1)
    c1, c2, c3 = _split3(cs)
    lhs = jnp.concatenate([c1, c1, c2, c1, c2, c3], axis=1)
    scores = lax.dot_general(
        lhs, rhs_s[...], (((1,), (0,)), ((), ())),
        preferred_element_type=f32,
    )

    pa = jnp.concatenate(
        [n_ref[...], -jnp.ones((TN, 1), f32), jnp.zeros((TN, 4), f32)], axis=1)
    p1 = pa.astype(jnp.bfloat16)
    p2 = (pa - p1.astype(f32)).astype(jnp.bfloat16)
    pl_lhs = jnp.concatenate([p1, p1, p2], axis=1)
    svals = lax.dot_general(
        pl_lhs, dns_s[...], (((1,), (0,)), ((), ())),
        preferred_element_type=f32,
    )

    rowmin = jnp.min(scores, axis=1, keepdims=True)
    eq = scores == rowmin
    dist = jnp.min(jnp.where(eq, svals, jnp.float32(1e30)), axis=1)
    pen = jnp.maximum(EPS - dist, 0.0)
    contrib = jnp.sum(pen * pen * pen)

    @pl.when(i == 0)
    def _():
        o_ref[...] = jnp.zeros((1, 1), jnp.float32)

    o_ref[...] += contrib.reshape(1, 1)


_penalty_call = pl.pallas_call(
    _tc_body,
    grid=(N // TN,),
    in_specs=[
        pl.BlockSpec((TN, 3), lambda i: (i, 0)),
        pl.BlockSpec((TN, 3), lambda i: (i, 0)),
        pl.BlockSpec((8, F), lambda i: (0, 0)),
        pl.BlockSpec((8, F), lambda i: (0, 0)),
    ],
    out_specs=pl.BlockSpec((1, 1), lambda i: (0, 0)),
    out_shape=jax.ShapeDtypeStruct((1, 1), jnp.float32),
    scratch_shapes=[
        pltpu.VMEM((48, F), jnp.bfloat16),
        pltpu.VMEM((24, F), jnp.bfloat16),
    ],
)


def kernel(next_pos, curr_pos, obstacle_next_pos, obstacle_curr_pos, obstacle_faces):
    faces = obstacle_faces.astype(jnp.int32)
    a_t, d_t = _face_tables(
        obstacle_curr_pos.T.reshape(-1),
        obstacle_next_pos.T.reshape(-1),
        faces.T.reshape(-1),
    )
    a_mat = a_t.reshape(NW, 8, FPT).transpose(1, 0, 2).reshape(8, F)
    d_mat = d_t.reshape(NW, 8, FPT).transpose(1, 0, 2).reshape(8, F)
    out = _penalty_call(curr_pos, next_pos, a_mat, d_mat)
    return out[0, 0]
